# baseline (device time: 145180 ns/iter reference)
import jax
import jax.numpy as jnp
from jax import lax
from jax.experimental import pallas as pl
from jax.experimental.pallas import tpu as pltpu

V_CHUNK = 512
BAND = 64


def kernel(x, W):
    t, d = x.shape
    _, v_loc = W.shape
    v_tot = 2 * v_loc
    ncc = v_loc // V_CHUNK
    q_cols = v_loc // 4
    nb = t // BAND

    def body(x_ref, w_hbm, out_hbm, w_buf, e_my, s_buf, s_in, cstage,
             res_buf, w_sems, cs_sems, rs_sems, st_sems,
             z_send_sems, z_recv_sems, x_send_sems, x_recv_sems,
             y_send_sems, y_recv_sems, s_send, s_recv):
        my_x = lax.axis_index("x")
        my_y = lax.axis_index("y")
        my_z = lax.axis_index("z")
        z_nbr = (my_x, my_y, 1 - my_z)
        x_nbr = (1 - my_x, my_y, my_z)
        y_nbr = (my_x, 1 - my_y, my_z)
        my_lo = my_z * v_loc
        other_lo = (1 - my_z) * v_loc

        qid_me = 2 * my_x + my_y
        qid_d = 2 * (1 - my_x) + (1 - my_y)
        qid_x = 2 * (1 - my_x) + my_y
        qid_y = 2 * my_x + (1 - my_y)

        def q_col(qid, j):
            return other_lo + qid * q_cols + j * V_CHUNK

        barrier = pltpu.get_barrier_semaphore()
        for nbr in (z_nbr, x_nbr, y_nbr):
            pl.semaphore_signal(
                barrier, inc=1, device_id=nbr,
                device_id_type=pl.DeviceIdType.MESH)
        pl.semaphore_wait(barrier, 3)

        chunk_ids = []
        for q in (qid_me, qid_d, qid_x, qid_y):
            for k in range(4):
                chunk_ids.append(q * 4 + k)

        def w_load(i):
            ci = chunk_ids[i]
            cp = pltpu.make_async_copy(
                w_hbm.at[:, pl.ds(ci * V_CHUNK, V_CHUNK)],
                w_buf.at[i % 2], w_sems.at[i % 2])
            cp.start()
            return cp

        z_rdmas = []
        pending_w = w_load(0)
        s_run = jnp.zeros((t, 1), jnp.float32)
        for i in range(ncc):
            ci = chunk_ids[i]
            pending_w.wait()
            if i + 1 < ncc:
                pending_w = w_load(i + 1)
            lc = lax.dot_general(
                x_ref[:, :], w_buf[i % 2],
                dimension_numbers=(((1,), (0,)), ((), ())),
                preferred_element_type=jnp.float32)
            e = jnp.exp(lc)
            e_my[:, pl.ds(ci * V_CHUNK, V_CHUNK)] = e
            s_run = s_run + jnp.sum(e, axis=1, keepdims=True)
            if i < 4:
                rdma = pltpu.make_async_remote_copy(
                    src_ref=e_my.at[:, pl.ds(ci * V_CHUNK, V_CHUNK)],
                    dst_ref=out_hbm.at[:, pl.ds(my_lo + ci * V_CHUNK,
                                                V_CHUNK)],
                    send_sem=z_send_sems.at[i],
                    recv_sem=z_recv_sems.at[i],
                    device_id=z_nbr,
                    device_id_type=pl.DeviceIdType.MESH)
                rdma.start()
                z_rdmas.append(rdma)

        s_buf[:, :] = s_run
        s_rdma = pltpu.make_async_remote_copy(
            src_ref=s_buf, dst_ref=s_in,
            send_sem=s_send, recv_sem=s_recv,
            device_id=z_nbr, device_id_type=pl.DeviceIdType.MESH)
        s_rdma.start()

        def plane_rdma(sems_pair, j, abs_col, nbr):
            send_sems_, recv_sems_ = sems_pair
            return pltpu.make_async_remote_copy(
                src_ref=out_hbm.at[:, pl.ds(abs_col, V_CHUNK)],
                dst_ref=out_hbm.at[:, pl.ds(abs_col, V_CHUNK)],
                send_sem=send_sems_.at[j], recv_sem=recv_sems_.at[j],
                device_id=nbr, device_id_type=pl.DeviceIdType.MESH)

        xp = (x_send_sems, x_recv_sems)
        yp = (y_send_sems, y_recv_sems)

        x_rdmas = []
        y_rdmas = []
        for j in range(4):
            z_rdmas[j].wait_recv()
            abs_col = q_col(qid_me, j)
            r = plane_rdma(xp, j, abs_col, x_nbr)
            r.start()
            x_rdmas.append(r)
            r = plane_rdma(yp, j, abs_col, y_nbr)
            r.start()
            y_rdmas.append(r)

        relay_to_y = []
        relay_to_x = []
        for j in range(4):
            plane_rdma(xp, j, q_col(qid_x, j), x_nbr).wait_recv()
            if j >= 2:
                r = plane_rdma(yp, 4 + (j - 2), q_col(qid_x, j), y_nbr)
                r.start()
                relay_to_y.append((j, r))
            plane_rdma(yp, j, q_col(qid_y, j), y_nbr).wait_recv()
            if j < 2:
                r = plane_rdma(xp, 4 + j, q_col(qid_y, j), x_nbr)
                r.start()
                relay_to_x.append((j, r))

        s_rdma.wait_recv()
        inv = 1.0 / (s_run + s_in[:, :])

        wb_slot = [None, None]
        piece_idx = 0

        def process_piece(abs_col, guard=None):
            nonlocal piece_idx
            sl = piece_idx % 2
            if wb_slot[sl] is not None:
                wb_slot[sl].wait()
                wb_slot[sl] = None
            cp = pltpu.make_async_copy(
                out_hbm.at[:, pl.ds(abs_col, V_CHUNK)],
                cstage.at[sl], cs_sems.at[sl])
            cp.start()
            cp.wait()
            cstage[sl, :, :] = cstage[sl] * inv
            if guard is not None:
                guard()
            wb = pltpu.make_async_copy(
                cstage.at[sl],
                out_hbm.at[:, pl.ds(abs_col, V_CHUNK)], st_sems.at[sl])
            wb.start()
            wb_slot[sl] = wb
            piece_idx += 1

        for j in range(4):
            xr, yr = x_rdmas[j], y_rdmas[j]
            process_piece(
                q_col(qid_me, j),
                guard=lambda xr=xr, yr=yr: (xr.wait_send(),
                                            yr.wait_send()))
        relay_y_guard = {j: r for j, r in relay_to_y}
        for j in range(4):
            g = relay_y_guard.get(j)
            process_piece(
                q_col(qid_x, j),
                guard=(lambda g=g: g.wait_send()) if g is not None
                else None)
        relay_x_guard = {j: r for j, r in relay_to_x}
        for j in range(4):
            g = relay_x_guard.get(j)
            process_piece(
                q_col(qid_y, j),
                guard=(lambda g=g: g.wait_send()) if g is not None
                else None)

        band_wb = [None, None]
        for b in range(nb):
            rows = slice(b * BAND, (b + 1) * BAND)
            if band_wb[b % 2] is not None:
                band_wb[b % 2].wait()
            res_buf[b % 2, :, :] = e_my[rows, :] * inv[rows]
            cp = pltpu.make_async_copy(
                res_buf.at[b % 2],
                out_hbm.at[pl.ds(b * BAND, BAND), pl.ds(my_lo, v_loc)],
                rs_sems.at[b % 2])
            cp.start()
            band_wb[b % 2] = cp

        for j in range(2):
            plane_rdma(xp, 4 + j, q_col(qid_d, j), x_nbr).wait_recv()
            process_piece(q_col(qid_d, j))
        for j in range(2):
            plane_rdma(yp, 4 + j, q_col(qid_d, 2 + j), y_nbr).wait_recv()
            process_piece(q_col(qid_d, 2 + j))

        for cp in band_wb + wb_slot:
            if cp is not None:
                cp.wait()
        for rdma in z_rdmas:
            rdma.wait_send()
        s_rdma.wait_send()

    return pl.pallas_call(
        body,
        out_shape=jax.ShapeDtypeStruct((t, v_tot), jnp.float32),
        in_specs=[
            pl.BlockSpec(memory_space=pltpu.VMEM),
            pl.BlockSpec(memory_space=pl.ANY),
        ],
        out_specs=pl.BlockSpec(memory_space=pl.ANY),
        scratch_shapes=[
            pltpu.VMEM((2, d, V_CHUNK), jnp.float32),
            pltpu.VMEM((t, v_loc), jnp.float32),
            pltpu.VMEM((t, 1), jnp.float32),
            pltpu.VMEM((t, 1), jnp.float32),
            pltpu.VMEM((2, t, V_CHUNK), jnp.float32),
            pltpu.VMEM((2, BAND, v_loc), jnp.float32),
            pltpu.SemaphoreType.DMA((2,)),
            pltpu.SemaphoreType.DMA((2,)),
            pltpu.SemaphoreType.DMA((2,)),
            pltpu.SemaphoreType.DMA((2,)),
            pltpu.SemaphoreType.DMA((4,)),
            pltpu.SemaphoreType.DMA((4,)),
            pltpu.SemaphoreType.DMA((6,)),
            pltpu.SemaphoreType.DMA((6,)),
            pltpu.SemaphoreType.DMA((6,)),
            pltpu.SemaphoreType.DMA((6,)),
            pltpu.SemaphoreType.DMA,
            pltpu.SemaphoreType.DMA,
        ],
        compiler_params=pltpu.CompilerParams(collective_id=0),
    )(x, W)


# device time: 127843 ns/iter; 1.1356x vs baseline; 1.1356x over previous
import jax
import jax.numpy as jnp
from jax import lax
from jax.experimental import pallas as pl
from jax.experimental.pallas import tpu as pltpu

V_CHUNK = 512
P_COLS = 1024
BAND = 64


def kernel(x, W):
    t, d = x.shape
    _, v_loc = W.shape
    v_tot = 2 * v_loc
    ncc = v_loc // V_CHUNK
    q_cols = v_loc // 4
    nb = t // BAND

    def body(x_ref, w_hbm, out_hbm, w_buf, e_my, s_buf, s_in, cstage,
             res_buf, w_sems, cs_sems, rs_sems, st_sems,
             z_send_sems, z_recv_sems, x_send_sems, x_recv_sems,
             y_send_sems, y_recv_sems, s_send, s_recv):
        my_x = lax.axis_index("x")
        my_y = lax.axis_index("y")
        my_z = lax.axis_index("z")
        z_nbr = (my_x, my_y, 1 - my_z)
        x_nbr = (1 - my_x, my_y, my_z)
        y_nbr = (my_x, 1 - my_y, my_z)
        my_lo = my_z * v_loc
        other_lo = (1 - my_z) * v_loc

        qid_me = 2 * my_x + my_y
        qid_d = 2 * (1 - my_x) + (1 - my_y)
        qid_x = 2 * (1 - my_x) + my_y
        qid_y = 2 * my_x + (1 - my_y)

        def q_col(qid, j):
            return other_lo + qid * q_cols + j * V_CHUNK

        barrier = pltpu.get_barrier_semaphore()
        for nbr in (z_nbr, x_nbr, y_nbr):
            pl.semaphore_signal(
                barrier, inc=1, device_id=nbr,
                device_id_type=pl.DeviceIdType.MESH)
        pl.semaphore_wait(barrier, 3)

        chunk_ids = []
        for q in (qid_me, qid_d, qid_x, qid_y):
            for k in range(4):
                chunk_ids.append(q * 4 + k)

        def w_load(i):
            ci = chunk_ids[i]
            cp = pltpu.make_async_copy(
                w_hbm.at[:, pl.ds(ci * V_CHUNK, V_CHUNK)],
                w_buf.at[i % 2], w_sems.at[i % 2])
            cp.start()
            return cp

        def z_send(j, ci):
            rdma = pltpu.make_async_remote_copy(
                src_ref=e_my.at[:, pl.ds(ci * V_CHUNK, V_CHUNK)],
                dst_ref=out_hbm.at[:, pl.ds(my_lo + ci * V_CHUNK,
                                            V_CHUNK)],
                send_sem=z_send_sems.at[j], recv_sem=z_recv_sems.at[j],
                device_id=z_nbr, device_id_type=pl.DeviceIdType.MESH)
            rdma.start()
            return rdma

        z_rdmas = []
        pending_w = w_load(0)
        s_run = jnp.zeros((t, 1), jnp.float32)
        for i in range(ncc):
            ci = chunk_ids[i]
            pending_w.wait()
            if i + 1 < ncc:
                pending_w = w_load(i + 1)
            lc = lax.dot_general(
                x_ref[:, :], w_buf[i % 2],
                dimension_numbers=(((1,), (0,)), ((), ())),
                preferred_element_type=jnp.float32)
            e = jnp.exp(lc)
            e_my[:, pl.ds(ci * V_CHUNK, V_CHUNK)] = e
            s_run = s_run + jnp.sum(e, axis=1, keepdims=True)
            if i < 4:
                z_rdmas.append(z_send(i, ci))

        s_buf[:, :] = s_run
        s_rdma = pltpu.make_async_remote_copy(
            src_ref=s_buf, dst_ref=s_in,
            send_sem=s_send, recv_sem=s_recv,
            device_id=z_nbr, device_id_type=pl.DeviceIdType.MESH)
        s_rdma.start()

        for j in range(4):
            z_rdmas.append(z_send(4 + j, chunk_ids[4 + j]))

        def plane_rdma(sems_pair, f, abs_col, nbr):
            send_sems_, recv_sems_ = sems_pair
            return pltpu.make_async_remote_copy(
                src_ref=out_hbm.at[:, pl.ds(abs_col, P_COLS)],
                dst_ref=out_hbm.at[:, pl.ds(abs_col, P_COLS)],
                send_sem=send_sems_.at[f], recv_sem=recv_sems_.at[f],
                device_id=nbr, device_id_type=pl.DeviceIdType.MESH)

        xp = (x_send_sems, x_recv_sems)
        yp = (y_send_sems, y_recv_sems)
        x_fwd = []
        y_fwd = []
        for f in range(2):
            z_rdmas[2 * f].wait_recv()
            z_rdmas[2 * f + 1].wait_recv()
            abs_col = q_col(qid_me, 2 * f)
            r = plane_rdma(xp, f, abs_col, x_nbr)
            r.start()
            x_fwd.append(r)
            r = plane_rdma(yp, f, abs_col, y_nbr)
            r.start()
            y_fwd.append(r)

        s_rdma.wait_recv()
        inv = 1.0 / (s_run + s_in[:, :])

        def run_pieces(pieces):
            n = len(pieces)
            stages = [None, None]
            wbs = [None, None]

            def start_stage(k):
                col, w, ready, _ = pieces[k]
                if ready is not None:
                    ready()
                sl = k % 2
                if wbs[sl] is not None:
                    wbs[sl].wait()
                    wbs[sl] = None
                cp = pltpu.make_async_copy(
                    out_hbm.at[:, pl.ds(col, w)],
                    cstage.at[sl, :, pl.ds(0, w)], cs_sems.at[sl])
                cp.start()
                stages[sl] = cp

            start_stage(0)
            for k in range(n):
                col, w, _, guard = pieces[k]
                sl = k % 2
                stages[sl].wait()
                stages[sl] = None
                if k + 1 < n:
                    start_stage(k + 1)
                cstage[sl, :, 0:w] = cstage[sl, :, 0:w] * inv
                if guard is not None:
                    guard()
                wb = pltpu.make_async_copy(
                    cstage.at[sl, :, pl.ds(0, w)],
                    out_hbm.at[:, pl.ds(col, w)], st_sems.at[sl])
                wb.start()
                wbs[sl] = wb
            for wb in wbs:
                if wb is not None:
                    wb.wait()

        pieces = []
        for f in range(2):
            xr, yr = x_fwd[f], y_fwd[f]
            pieces.append((
                q_col(qid_me, 2 * f), P_COLS, None,
                lambda xr=xr, yr=yr: (xr.wait_send(), yr.wait_send())))
        for f in range(2):
            pieces.append((
                q_col(qid_x, 2 * f), P_COLS,
                lambda f=f: plane_rdma(
                    xp, f, q_col(qid_x, 2 * f), x_nbr).wait_recv(),
                None))
        for f in range(2):
            pieces.append((
                q_col(qid_y, 2 * f), P_COLS,
                lambda f=f: plane_rdma(
                    yp, f, q_col(qid_y, 2 * f), y_nbr).wait_recv(),
                None))
        run_pieces(pieces)

        band_wb = [None, None]
        for b in range(nb):
            rows = slice(b * BAND, (b + 1) * BAND)
            if band_wb[b % 2] is not None:
                band_wb[b % 2].wait()
            res_buf[b % 2, :, :] = e_my[rows, :] * inv[rows]
            cp = pltpu.make_async_copy(
                res_buf.at[b % 2],
                out_hbm.at[pl.ds(b * BAND, BAND), pl.ds(my_lo, v_loc)],
                rs_sems.at[b % 2])
            cp.start()
            band_wb[b % 2] = cp

        run_pieces([
            (q_col(qid_d, j), V_CHUNK,
             lambda j=j: z_rdmas[4 + j].wait_recv(), None)
            for j in range(4)
        ])

        for cp in band_wb:
            if cp is not None:
                cp.wait()
        for rdma in z_rdmas:
            rdma.wait_send()
        s_rdma.wait_send()

    return pl.pallas_call(
        body,
        out_shape=jax.ShapeDtypeStruct((t, v_tot), jnp.float32),
        in_specs=[
            pl.BlockSpec(memory_space=pltpu.VMEM),
            pl.BlockSpec(memory_space=pl.ANY),
        ],
        out_specs=pl.BlockSpec(memory_space=pl.ANY),
        scratch_shapes=[
            pltpu.VMEM((2, d, V_CHUNK), jnp.float32),
            pltpu.VMEM((t, v_loc), jnp.float32),
            pltpu.VMEM((t, 1), jnp.float32),
            pltpu.VMEM((t, 1), jnp.float32),
            pltpu.VMEM((2, t, P_COLS), jnp.float32),
            pltpu.VMEM((2, BAND, v_loc), jnp.float32),
            pltpu.SemaphoreType.DMA((2,)),
            pltpu.SemaphoreType.DMA((2,)),
            pltpu.SemaphoreType.DMA((2,)),
            pltpu.SemaphoreType.DMA((2,)),
            pltpu.SemaphoreType.DMA((8,)),
            pltpu.SemaphoreType.DMA((8,)),
            pltpu.SemaphoreType.DMA((2,)),
            pltpu.SemaphoreType.DMA((2,)),
            pltpu.SemaphoreType.DMA((2,)),
            pltpu.SemaphoreType.DMA((2,)),
            pltpu.SemaphoreType.DMA,
            pltpu.SemaphoreType.DMA,
        ],
        compiler_params=pltpu.CompilerParams(collective_id=0),
    )(x, W)


# device time: 89035 ns/iter; 1.6306x vs baseline; 1.4359x over previous
import jax
import jax.numpy as jnp
from jax import lax
from jax.experimental import pallas as pl
from jax.experimental.pallas import tpu as pltpu

V_CHUNK = 512
P_COLS = 1024
BAND = 64


def kernel(x, W):
    t, d = x.shape
    _, v_loc = W.shape
    v_tot = 2 * v_loc
    ncc = v_loc // V_CHUNK
    q_cols = v_loc // 4
    nb = t // BAND

    def body(x_ref, w_hbm, out_hbm, w_buf, e_bf, in_bf, s_buf, s_in,
             res_p, res_b, w_sems, rp_sems, rb_sems,
             z_send_sems, z_recv_sems, x_send_sems, x_recv_sems,
             y_send_sems, y_recv_sems, s_send, s_recv):
        my_x = lax.axis_index("x")
        my_y = lax.axis_index("y")
        my_z = lax.axis_index("z")
        z_nbr = (my_x, my_y, 1 - my_z)
        x_nbr = (1 - my_x, my_y, my_z)
        y_nbr = (my_x, 1 - my_y, my_z)
        my_lo = my_z * v_loc
        other_lo = (1 - my_z) * v_loc

        qid_me = 2 * my_x + my_y
        qid_d = 2 * (1 - my_x) + (1 - my_y)
        qid_x = 2 * (1 - my_x) + my_y
        qid_y = 2 * my_x + (1 - my_y)

        barrier = pltpu.get_barrier_semaphore()
        for nbr in (z_nbr, x_nbr, y_nbr):
            pl.semaphore_signal(
                barrier, inc=1, device_id=nbr,
                device_id_type=pl.DeviceIdType.MESH)
        pl.semaphore_wait(barrier, 3)

        chunk_ids = []
        for q in (qid_me, qid_d, qid_x, qid_y):
            for k in range(4):
                chunk_ids.append(q * 4 + k)

        def w_load(i):
            ci = chunk_ids[i]
            cp = pltpu.make_async_copy(
                w_hbm.at[:, pl.ds(ci * V_CHUNK, V_CHUNK)],
                w_buf.at[i % 2], w_sems.at[i % 2])
            cp.start()
            return cp

        def z_send(j, ci):
            rdma = pltpu.make_async_remote_copy(
                src_ref=e_bf.at[:, pl.ds(ci * V_CHUNK, V_CHUNK)],
                dst_ref=in_bf.at[:, pl.ds(ci * V_CHUNK, V_CHUNK)],
                send_sem=z_send_sems.at[j], recv_sem=z_recv_sems.at[j],
                device_id=z_nbr, device_id_type=pl.DeviceIdType.MESH)
            rdma.start()
            return rdma

        z_rdmas = []
        pending_w = w_load(0)
        s_run = jnp.zeros((t, 1), jnp.float32)
        for i in range(ncc):
            ci = chunk_ids[i]
            pending_w.wait()
            if i + 1 < ncc:
                pending_w = w_load(i + 1)
            lc = lax.dot_general(
                x_ref[:, :], w_buf[i % 2],
                dimension_numbers=(((1,), (0,)), ((), ())),
                preferred_element_type=jnp.float32)
            e = jnp.exp(lc)
            e_bf[:, pl.ds(ci * V_CHUNK, V_CHUNK)] = e.astype(jnp.bfloat16)
            s_run = s_run + jnp.sum(e, axis=1, keepdims=True)
            if i < 4:
                z_rdmas.append(z_send(i, ci))

        s_buf[:, :] = s_run
        s_rdma = pltpu.make_async_remote_copy(
            src_ref=s_buf, dst_ref=s_in,
            send_sem=s_send, recv_sem=s_recv,
            device_id=z_nbr, device_id_type=pl.DeviceIdType.MESH)
        s_rdma.start()

        for j in range(4):
            z_rdmas.append(z_send(4 + j, chunk_ids[4 + j]))

        def plane_rdma(sems_pair, f, col, nbr):
            send_sems_, recv_sems_ = sems_pair
            return pltpu.make_async_remote_copy(
                src_ref=in_bf.at[:, pl.ds(col, P_COLS)],
                dst_ref=in_bf.at[:, pl.ds(col, P_COLS)],
                send_sem=send_sems_.at[f], recv_sem=recv_sems_.at[f],
                device_id=nbr, device_id_type=pl.DeviceIdType.MESH)

        xp = (x_send_sems, x_recv_sems)
        yp = (y_send_sems, y_recv_sems)
        x_fwd = []
        y_fwd = []
        for f in range(2):
            z_rdmas[2 * f].wait_recv()
            z_rdmas[2 * f + 1].wait_recv()
            col = qid_me * q_cols + f * P_COLS
            r = plane_rdma(xp, f, col, x_nbr)
            r.start()
            x_fwd.append(r)
            r = plane_rdma(yp, f, col, y_nbr)
            r.start()
            y_fwd.append(r)

        s_rdma.wait_recv()
        inv = 1.0 / (s_run + s_in[:, :])

        wbs = [None, None]
        piece_idx = 0

        def process_piece(col, w, ready):
            nonlocal piece_idx
            if ready is not None:
                ready()
            sl = piece_idx % 2
            if wbs[sl] is not None:
                wbs[sl].wait()
                wbs[sl] = None
            res_p[sl, :, pl.ds(0, w)] = (
                in_bf[:, pl.ds(col, w)].astype(jnp.float32) * inv)
            wb = pltpu.make_async_copy(
                res_p.at[sl, :, pl.ds(0, w)],
                out_hbm.at[:, pl.ds(other_lo + col, w)], rp_sems.at[sl])
            wb.start()
            wbs[sl] = wb
            piece_idx += 1

        for f in range(2):
            process_piece(qid_me * q_cols + f * P_COLS, P_COLS, None)
        for f in range(2):
            process_piece(
                qid_x * q_cols + f * P_COLS, P_COLS,
                lambda f=f: plane_rdma(
                    xp, f, qid_x * q_cols + f * P_COLS, x_nbr
                ).wait_recv())
        for f in range(2):
            process_piece(
                qid_y * q_cols + f * P_COLS, P_COLS,
                lambda f=f: plane_rdma(
                    yp, f, qid_y * q_cols + f * P_COLS, y_nbr
                ).wait_recv())

        band_wb = [None, None]
        for b in range(nb):
            rows = slice(b * BAND, (b + 1) * BAND)
            if band_wb[b % 2] is not None:
                band_wb[b % 2].wait()
            res_b[b % 2, :, :] = (
                e_bf[rows, :].astype(jnp.float32) * inv[rows])
            cp = pltpu.make_async_copy(
                res_b.at[b % 2],
                out_hbm.at[pl.ds(b * BAND, BAND), pl.ds(my_lo, v_loc)],
                rb_sems.at[b % 2])
            cp.start()
            band_wb[b % 2] = cp

        for j in range(4):
            process_piece(
                qid_d * q_cols + j * V_CHUNK, V_CHUNK,
                lambda j=j: z_rdmas[4 + j].wait_recv())

        for cp in band_wb + wbs:
            if cp is not None:
                cp.wait()
        for rdma in z_rdmas + x_fwd + y_fwd:
            rdma.wait_send()
        s_rdma.wait_send()

    return pl.pallas_call(
        body,
        out_shape=jax.ShapeDtypeStruct((t, v_tot), jnp.float32),
        in_specs=[
            pl.BlockSpec(memory_space=pltpu.VMEM),
            pl.BlockSpec(memory_space=pl.ANY),
        ],
        out_specs=pl.BlockSpec(memory_space=pl.ANY),
        scratch_shapes=[
            pltpu.VMEM((2, d, V_CHUNK), jnp.float32),
            pltpu.VMEM((t, v_loc), jnp.bfloat16),
            pltpu.VMEM((t, v_loc), jnp.bfloat16),
            pltpu.VMEM((t, 1), jnp.float32),
            pltpu.VMEM((t, 1), jnp.float32),
            pltpu.VMEM((2, t, P_COLS), jnp.float32),
            pltpu.VMEM((2, BAND, v_loc), jnp.float32),
            pltpu.SemaphoreType.DMA((2,)),
            pltpu.SemaphoreType.DMA((2,)),
            pltpu.SemaphoreType.DMA((2,)),
            pltpu.SemaphoreType.DMA((8,)),
            pltpu.SemaphoreType.DMA((8,)),
            pltpu.SemaphoreType.DMA((2,)),
            pltpu.SemaphoreType.DMA((2,)),
            pltpu.SemaphoreType.DMA((2,)),
            pltpu.SemaphoreType.DMA((2,)),
            pltpu.SemaphoreType.DMA,
            pltpu.SemaphoreType.DMA,
        ],
        compiler_params=pltpu.CompilerParams(collective_id=0),
    )(x, W)
